# Initial kernel scaffold; baseline (speedup 1.0000x reference)
import jax
import jax.numpy as jnp
from jax.experimental import pallas as pl


def kernel(x, edge_index, edge_attr, batch, demographics, emb, W0_0, W0_1, W0_2, b0, W1_0, W1_1, W1_2, b1, W2_0, W2_1, W2_2, b2, Wc1, bc1, Wc2, bc2):
    raise NotImplementedError("write your pallas kernel here")


# speedup vs baseline: 1.7096x; 1.7096x over previous
import functools
import jax
import jax.numpy as jnp
from jax import lax
from jax.experimental import pallas as pl
from jax.experimental.pallas import tpu as pltpu
from jax.experimental.pallas import tpu_sc as plsc

NWORK = 32
CH = 80
F32 = jnp.float32


def _mesh():
    return plsc.VectorSubcoreMesh(core_axis_name="c", subcore_axis_name="s")


def _gather16(table, idx):
    M = idx.shape[0]
    per_w = M // NWORK
    iters = per_w // CH

    @functools.partial(
        pl.kernel,
        mesh=_mesh(),
        out_type=jax.ShapeDtypeStruct((M, 16), F32),
        scratch_types=[
            pltpu.VMEM((CH,), jnp.int32),
            pltpu.VMEM((CH, 16), F32),
            pltpu.SemaphoreType.DMA,
        ],
        compiler_params=pltpu.CompilerParams(use_tc_tiling_on_sc=False),
    )
    def k(table_hbm, idx_hbm, out_hbm, idx_v, rows_v, sem):
        wid = lax.axis_index("s") * 2 + lax.axis_index("c")
        base = wid * per_w

        def body(i, carry):
            off = base + i * CH
            pltpu.sync_copy(idx_hbm.at[pl.ds(off, CH)], idx_v)
            pltpu.async_copy(table_hbm.at[idx_v], rows_v, sem).wait()
            pltpu.sync_copy(rows_v, out_hbm.at[pl.ds(off, CH)])
            return carry

        lax.fori_loop(0, iters, body, 0)

    return k(table, idx)


def _scatter_add16(values, idx, zeros_acc):
    M = idx.shape[0]
    R = zeros_acc.shape[0]
    per_w = M // NWORK
    iters = per_w // CH
    rpw = R // 16

    @functools.partial(
        pl.kernel,
        mesh=_mesh(),
        out_type=jax.ShapeDtypeStruct((2, R, 16), F32),
        scratch_types=[
            pltpu.VMEM((CH,), jnp.int32),
            pltpu.VMEM((CH, 16), F32),
            pltpu.VMEM_SHARED((R, 16), F32),
        ],
        compiler_params=pltpu.CompilerParams(use_tc_tiling_on_sc=False),
    )
    def k(val_hbm, idx_hbm, zero_hbm, out_hbm, idx_v, val_v, acc_sh):
        cid = lax.axis_index("c")
        sid = lax.axis_index("s")
        wid = sid * 2 + cid

        @pl.when(sid == 0)
        def _init():
            pltpu.sync_copy(zero_hbm, acc_sh)

        plsc.subcore_barrier()

        def body(i, carry):
            off = wid * per_w + i * CH
            pltpu.sync_copy(idx_hbm.at[pl.ds(off, CH)], idx_v)
            pltpu.sync_copy(val_hbm.at[pl.ds(off, CH)], val_v)
            pltpu.sync_copy(val_v, acc_sh.at[idx_v], add=True)
            return carry

        lax.fori_loop(0, iters, body, 0)
        plsc.subcore_barrier()

        @pl.when(sid == 0)
        def _drain():
            pltpu.sync_copy(acc_sh, out_hbm.at[cid])

    return k(values, idx, zeros_acc)


def _ew16(f, *arrs):
    M = arrs[0].shape[0]
    mb = 5000
    grid = M // mb

    def body(*refs):
        out = refs[-1]
        out[...] = f(*[r[...] for r in refs[:-1]])

    spec = pl.BlockSpec((mb, 16), lambda i: (i, 0))
    return pl.pallas_call(
        body,
        grid=(grid,),
        in_specs=[spec] * len(arrs),
        out_specs=spec,
        out_shape=jax.ShapeDtypeStruct((M, 16), F32),
    )(*arrs)


def _combine(t0_chunks, t1_chunks, p2_chunks, W0, W1, W2, b):
    N = t0_chunks[0].shape[0]
    nc = len(t0_chunks)
    fout = W0.shape[1]
    nb = 2000
    grid = N // nb

    def body(*refs):
        ts = [r[...] for r in refs[:nc]]
        t1s = [r[...] for r in refs[nc:2 * nc]]
        p2s = [r[...] for r in refs[2 * nc:3 * nc]]
        w0, w1, w2, bb = [r[...] for r in refs[3 * nc:3 * nc + 4]]
        out = refs[-1]
        T0 = jnp.concatenate(ts, axis=1)
        T1 = jnp.concatenate(t1s, axis=1)
        T2 = 2.0 * jnp.concatenate(p2s, axis=1) - T0
        acc = jnp.dot(T0, w0, preferred_element_type=F32)
        acc += jnp.dot(T1, w1, preferred_element_type=F32)
        acc += jnp.dot(T2, w2, preferred_element_type=F32)
        out[...] = acc + bb[0]

    nspec = pl.BlockSpec((nb, 16), lambda i: (i, 0))
    wspec = pl.BlockSpec(W0.shape, lambda i: (0, 0))
    bspec = pl.BlockSpec((1, fout), lambda i: (0, 0))
    return pl.pallas_call(
        body,
        grid=(grid,),
        in_specs=[nspec] * (3 * nc) + [wspec, wspec, wspec, bspec],
        out_specs=pl.BlockSpec((nb, fout), lambda i: (i, 0)),
        out_shape=jax.ShapeDtypeStruct((N, fout), F32),
    )(*t0_chunks, *t1_chunks, *p2_chunks, W0, W1, W2, b.reshape(1, -1))


def _counts(batch, B):
    N = batch.shape[0]
    cols = 100
    rows = N // cols
    rb = 40
    batch2 = batch.reshape(rows, cols)

    def body(b_ref, out_ref):
        i = pl.program_id(0)

        @pl.when(i == 0)
        def _():
            out_ref[...] = jnp.zeros_like(out_ref)

        seg = b_ref[...]
        eq = (seg[:, :, None] == lax.broadcasted_iota(jnp.int32, (1, 1, B), 2)
              ).astype(F32)
        out_ref[...] += jnp.sum(eq, axis=(0, 1), keepdims=False)[None]

    return pl.pallas_call(
        body,
        grid=(rows // rb,),
        in_specs=[pl.BlockSpec((rb, cols), lambda i: (i, 0))],
        out_specs=pl.BlockSpec((1, B), lambda i: (0, 0)),
        out_shape=jax.ShapeDtypeStruct((1, B), F32),
    )(batch2)


def _head(pool_parts, cnt, demographics, Wc1, bc1, Wc2, bc2, B):
    nchunk = len(pool_parts) // 2

    def body(*refs):
        parts = [r[...] for r in refs[:2 * nchunk]]
        cnt_r, demo, w1, b1r, w2, b2r = [r[...] for r in refs[2 * nchunk:-1]]
        out = refs[-1]
        sums = jnp.concatenate(
            [parts[2 * i] + parts[2 * i + 1] for i in range(nchunk)], axis=1
        )[:B]
        gf = sums / jnp.maximum(cnt_r.reshape(B, 1), 1.0)
        comb = jnp.concatenate([gf, demo], axis=1)
        z = jnp.maximum(jnp.dot(comb, w1, preferred_element_type=F32) + b1r[0], 0.0)
        out[...] = jnp.dot(z, w2, preferred_element_type=F32) + b2r[0]

    ins = list(pool_parts) + [cnt, demographics, Wc1, bc1.reshape(1, -1),
                              Wc2, bc2.reshape(1, -1)]
    return pl.pallas_call(
        body,
        in_specs=[pl.BlockSpec(a.shape, lambda: (0, 0)) for a in ins],
        out_specs=pl.BlockSpec((B, Wc2.shape[1]), lambda: (0, 0)),
        out_shape=jax.ShapeDtypeStruct((B, Wc2.shape[1]), F32),
    )(*ins)


def kernel(x, edge_index, edge_attr, batch, demographics, emb,
           W0_0, W0_1, W0_2, b0, W1_0, W1_1, W1_2, b1,
           W2_0, W2_1, W2_2, b2, Wc1, bc1, Wc2, bc2):
    N = x.shape[0]
    E = edge_index.shape[1]
    B = demographics.shape[0]

    row = edge_index[0].astype(jnp.int32)
    col = edge_index[1].astype(jnp.int32)
    ew = edge_attr[:, 0].astype(F32)
    ew16 = jnp.broadcast_to(ew[:, None], (E, 16))

    npad = (-N) % (NWORK * CH)
    Np = N + npad
    zeros_n = jnp.zeros((N, 16), F32)
    racc = 80
    zeros_b = jnp.zeros((racc, 16), F32)

    deg_p = _scatter_add16(ew16, row, zeros_n)
    dinv16 = _ew16(
        lambda a, b_: jnp.where(a + b_ > 0, lax.rsqrt(jnp.maximum(a + b_, 1e-30)), 0.0),
        deg_p[0], deg_p[1])
    grow = _gather16(dinv16, row)
    gcol = _gather16(dinv16, col)
    c16 = _ew16(lambda a, w, c_: -(a * w * c_), grow, ew16, gcol)

    xp = jnp.concatenate([x.astype(jnp.int32), jnp.zeros((npad,), jnp.int32)])
    h = _gather16(emb.astype(F32), xp)[:N]

    def prop_chunks(chunks):
        out = []
        for t in chunks:
            G = _gather16(t, row)
            S = _ew16(lambda a, b_: a * b_, c16, G)
            P = _scatter_add16(S, col, zeros_n)
            out.append(_ew16(lambda a, b_: a + b_, P[0], P[1]))
        return out

    def cheb(chunks, W0, W1, W2, bb):
        t1 = prop_chunks(chunks)
        p2 = prop_chunks(t1)
        full = _combine(chunks, t1, p2, W0, W1, W2, bb)
        fo = W0.shape[1]
        return [full[:, 16 * i:16 * (i + 1)] for i in range(fo // 16)]

    hc = cheb([h], W0_0, W0_1, W0_2, b0)
    hc = cheb(hc, W1_0, W1_1, W1_2, b1)
    hc = cheb(hc, W2_0, W2_1, W2_2, b2)

    batch_p = jnp.concatenate(
        [batch.astype(jnp.int32), jnp.full((npad,), B, jnp.int32)])
    pool_parts = []
    for t in hc:
        tp = jnp.concatenate([t, jnp.zeros((npad, 16), F32)], axis=0)
        pp = _scatter_add16(tp, batch_p, zeros_b)
        pool_parts += [pp[0], pp[1]]

    cnt = _counts(batch.astype(jnp.int32), B)
    return _head(pool_parts, cnt, demographics.astype(F32),
                 Wc1, bc1, Wc2, bc2, B)
